# Initial kernel scaffold; baseline (speedup 1.0000x reference)
import jax
import jax.numpy as jnp
from jax.experimental import pallas as pl


def kernel(x, edge_index, W1, b1, W2, b2):
    raise NotImplementedError("write your pallas kernel here")


# speedup vs baseline: 27.2357x; 27.2357x over previous
import functools

import jax
import jax.numpy as jnp
from jax import lax
from jax.experimental import pallas as pl
from jax.experimental.pallas import tpu as pltpu
from jax.experimental.pallas import tpu_sc as plsc

NC = 2
NS = 16
NW = NC * NS
C = 80


def _deg_kernel(n, ch):
    rpt = n // NS

    mesh = plsc.VectorSubcoreMesh(core_axis_name="c", subcore_axis_name="s")

    @functools.partial(
        pl.kernel,
        out_type=jax.ShapeDtypeStruct((NC, n, 16), jnp.float32),
        mesh=mesh,
        compiler_params=pltpu.CompilerParams(use_tc_tiling_on_sc=False),
        scratch_types=[
            pltpu.VMEM((ch, C), jnp.int32),
            pltpu.VMEM((C, 16), jnp.float32),
            pltpu.VMEM_SHARED((n, 16), jnp.float32),
        ],
    )
    def k(dst_hbm, zeros_hbm, ones_hbm, out_hbm, dst_v, ones_v, acc):
        c = lax.axis_index("c")
        s = lax.axis_index("s")
        wid = c * NS + s
        pltpu.sync_copy(dst_hbm.at[wid], dst_v)
        pltpu.sync_copy(ones_hbm, ones_v)
        pltpu.sync_copy(zeros_hbm.at[pl.ds(s * rpt, rpt)], acc.at[pl.ds(s * rpt, rpt)])
        plsc.subcore_barrier()

        def body(g, carry):
            pltpu.sync_copy(ones_v, acc.at[dst_v.at[g]], add=True)
            return carry

        lax.fori_loop(0, ch, body, 0)
        plsc.subcore_barrier()
        pltpu.sync_copy(acc.at[pl.ds(s * rpt, rpt)],
                        out_hbm.at[c, pl.ds(s * rpt, rpt)])

    return k


def _agg_kernel(n, ch):
    rpt = n // NS

    mesh = plsc.VectorSubcoreMesh(core_axis_name="c", subcore_axis_name="s")

    @functools.partial(
        pl.kernel,
        out_type=jax.ShapeDtypeStruct((NC, n, 16), jnp.float32),
        mesh=mesh,
        compiler_params=pltpu.CompilerParams(use_tc_tiling_on_sc=False),
        scratch_types=[
            pltpu.VMEM((ch, C), jnp.int32),
            pltpu.VMEM((ch, C), jnp.int32),
            pltpu.VMEM((C, 16), jnp.float32),
            pltpu.VMEM_SHARED((n, 16), jnp.float32),
            pltpu.SemaphoreType.DMA,
        ],
    )
    def k(hs_hbm, src_hbm, dst_hbm, zeros_hbm, out_hbm,
          src_v, dst_v, rows_v, acc, gsem):
        c = lax.axis_index("c")
        s = lax.axis_index("s")
        wid = c * NS + s
        pltpu.sync_copy(src_hbm.at[wid], src_v)
        pltpu.sync_copy(dst_hbm.at[wid], dst_v)
        pltpu.sync_copy(zeros_hbm.at[pl.ds(s * rpt, rpt)], acc.at[pl.ds(s * rpt, rpt)])
        plsc.subcore_barrier()

        def body(g, carry):
            pltpu.async_copy(hs_hbm.at[src_v.at[g]], rows_v, gsem).wait()
            pltpu.sync_copy(rows_v, acc.at[dst_v.at[g]], add=True)
            return carry

        lax.fori_loop(0, ch, body, 0)
        plsc.subcore_barrier()
        pltpu.sync_copy(acc.at[pl.ds(s * rpt, rpt)],
                        out_hbm.at[c, pl.ds(s * rpt, rpt)])

    return k


def _tc_first(x_ref, w1_ref, pdeg0_ref, pdeg1_ref, hs1_ref, dis_ref):
    p = pdeg0_ref[...] + pdeg1_ref[...]
    deg = p[:, 0:1] + 1.0
    dis = lax.rsqrt(deg)
    h1 = jnp.dot(x_ref[...], w1_ref[...], preferred_element_type=jnp.float32)
    hs1_ref[...] = h1 * dis
    dis_ref[...] = dis


def _tc_mid(p0_ref, p1_ref, hs1_ref, dis_ref, b1_ref, w2_ref, hs2_ref):
    dis = dis_ref[...]
    agg = (p0_ref[...] + p1_ref[...] + hs1_ref[...]) * dis + b1_ref[...]
    o1 = jnp.maximum(agg, 0.0)
    h2 = jnp.dot(o1, w2_ref[...], preferred_element_type=jnp.float32)
    hs2_ref[...] = h2 * dis


def _tc_last(p0_ref, p1_ref, hs2_ref, dis_ref, b2_ref, out_ref):
    z = (p0_ref[...] + p1_ref[...] + hs2_ref[...]) * dis_ref[...] + b2_ref[...]
    m = jnp.max(z, axis=1, keepdims=True)
    t = z - m
    lse = jnp.log(jnp.sum(jnp.exp(t), axis=1, keepdims=True))
    out_ref[...] = t - lse


def kernel(x, edge_index, W1, b1, W2, b2):
    n, _ = x.shape
    e = edge_index.shape[1]
    epw = e // NW
    ch = epw // C
    npad = ((n + 127) // 128) * 128

    src3 = edge_index[0].reshape(NW, ch, C)
    dst3 = edge_index[1].reshape(NW, ch, C)
    zeros_t = jnp.zeros((npad, 16), jnp.float32)
    ones_t = jnp.ones((C, 16), jnp.float32)

    pdeg = _deg_kernel(npad, ch)(dst3, zeros_t, ones_t)

    hs1, dis = pl.pallas_call(
        _tc_first,
        out_shape=(jax.ShapeDtypeStruct((n, 16), jnp.float32),
                   jax.ShapeDtypeStruct((n, 1), jnp.float32)),
    )(x, W1, pdeg[0, :n], pdeg[1, :n])

    agg = _agg_kernel(npad, ch)
    p1 = agg(hs1, src3, dst3, zeros_t)

    hs2 = pl.pallas_call(
        _tc_mid,
        out_shape=jax.ShapeDtypeStruct((n, 16), jnp.float32),
    )(p1[0, :n], p1[1, :n], hs1, dis, b1.reshape(1, 16), W2)

    p2 = agg(hs2, src3, dst3, zeros_t)

    out = pl.pallas_call(
        _tc_last,
        out_shape=jax.ShapeDtypeStruct((n, 16), jnp.float32),
    )(p2[0, :n], p2[1, :n], hs2, dis, b2.reshape(1, 16))

    return out


# speedup vs baseline: 54.1430x; 1.9879x over previous
import functools

import jax
import jax.numpy as jnp
from jax import lax
from jax.experimental import pallas as pl
from jax.experimental.pallas import tpu as pltpu
from jax.experimental.pallas import tpu_sc as plsc

NC = 2
NS = 16
NW = NC * NS
C = 100
NBUF = 5


def _deg_kernel(n, ch):
    rpt = n // NS

    mesh = plsc.VectorSubcoreMesh(core_axis_name="c", subcore_axis_name="s")

    @functools.partial(
        pl.kernel,
        out_type=jax.ShapeDtypeStruct((NC, n, 16), jnp.float32),
        mesh=mesh,
        compiler_params=pltpu.CompilerParams(use_tc_tiling_on_sc=False),
        scratch_types=[
            pltpu.VMEM((ch, C), jnp.int32),
            pltpu.VMEM((C, 16), jnp.float32),
            pltpu.VMEM_SHARED((n, 16), jnp.float32),
        ],
    )
    def k(dst_hbm, zeros_hbm, ones_hbm, out_hbm, dst_v, ones_v, acc):
        c = lax.axis_index("c")
        s = lax.axis_index("s")
        wid = c * NS + s
        pltpu.sync_copy(dst_hbm.at[wid], dst_v)
        pltpu.sync_copy(ones_hbm, ones_v)
        pltpu.sync_copy(zeros_hbm.at[pl.ds(s * rpt, rpt)], acc.at[pl.ds(s * rpt, rpt)])
        plsc.subcore_barrier()

        def body(g, carry):
            pltpu.sync_copy(ones_v, acc.at[dst_v.at[g]], add=True)
            return carry

        lax.fori_loop(0, ch, body, 0)
        plsc.subcore_barrier()
        pltpu.sync_copy(acc.at[pl.ds(s * rpt, rpt)],
                        out_hbm.at[c, pl.ds(s * rpt, rpt)])

    return k


def _agg_kernel(n, ch):
    rpt = n // NS

    mesh = plsc.VectorSubcoreMesh(core_axis_name="c", subcore_axis_name="s")

    @functools.partial(
        pl.kernel,
        out_type=jax.ShapeDtypeStruct((NC, n, 16), jnp.float32),
        mesh=mesh,
        compiler_params=pltpu.CompilerParams(use_tc_tiling_on_sc=False),
        scratch_types=[
            pltpu.VMEM((ch, C), jnp.int32),
            pltpu.VMEM((ch, C), jnp.int32),
            pltpu.VMEM((2, NBUF, C, 16), jnp.float32),
            pltpu.VMEM_SHARED((n, 16), jnp.float32),
            pltpu.SemaphoreType.DMA((2,)),
        ],
    )
    def k(hs_hbm, src_hbm, dst_hbm, zeros_hbm, out_hbm,
          src_v, dst_v, rows_v, acc, gsem):
        c = lax.axis_index("c")
        s = lax.axis_index("s")
        wid = c * NS + s
        pltpu.sync_copy(src_hbm.at[wid], src_v)
        pltpu.sync_copy(dst_hbm.at[wid], dst_v)
        pltpu.sync_copy(zeros_hbm.at[pl.ds(s * rpt, rpt)], acc.at[pl.ds(s * rpt, rpt)])
        plsc.subcore_barrier()

        ngroups = ch // NBUF
        npairs = ngroups // 2

        def fire(grp_idx, p):
            for b in range(NBUF):
                pltpu.async_copy(hs_hbm.at[src_v.at[grp_idx * NBUF + b]],
                                 rows_v.at[p, b], gsem.at[p])

        def drain_scatter(grp_idx, p):
            for b in range(NBUF):
                pltpu.make_async_copy(hs_hbm.at[src_v.at[grp_idx * NBUF + b]],
                                      rows_v.at[p, b], gsem.at[p]).wait()
            for b in range(NBUF):
                pltpu.sync_copy(rows_v.at[p, b],
                                acc.at[dst_v.at[grp_idx * NBUF + b]], add=True)

        fire(0, 0)
        fire(1, 1)

        def pair(k_, carry):
            for p in range(2):
                gi = 2 * k_ + p
                drain_scatter(gi, p)
                fire(gi + 2, p)
            return carry

        lax.fori_loop(0, npairs - 1, pair, 0)
        for p in range(2):
            drain_scatter(2 * (npairs - 1) + p, p)
        plsc.subcore_barrier()
        pltpu.sync_copy(acc.at[pl.ds(s * rpt, rpt)],
                        out_hbm.at[c, pl.ds(s * rpt, rpt)])

    return k


def _tc_first(x_ref, w1_ref, pdeg_ref, hs1_ref, dis_ref):
    n = x_ref.shape[0]
    p = pdeg_ref[0, :n, :] + pdeg_ref[1, :n, :]
    deg = p[:, 0:1] + 1.0
    dis = lax.rsqrt(deg)
    h1 = jnp.dot(x_ref[...], w1_ref[...], preferred_element_type=jnp.float32)
    hs1_ref[...] = h1 * dis
    dis_ref[...] = dis


def _tc_mid(p_ref, hs1_ref, dis_ref, b1_ref, w2_ref, hs2_ref):
    n = hs1_ref.shape[0]
    dis = dis_ref[...]
    agg = (p_ref[0, :n, :] + p_ref[1, :n, :] + hs1_ref[...]) * dis + b1_ref[...]
    o1 = jnp.maximum(agg, 0.0)
    h2 = jnp.dot(o1, w2_ref[...], preferred_element_type=jnp.float32)
    hs2_ref[...] = h2 * dis


def _tc_last(p_ref, hs2_ref, dis_ref, b2_ref, out_ref):
    n = hs2_ref.shape[0]
    z = (p_ref[0, :n, :] + p_ref[1, :n, :] + hs2_ref[...]) * dis_ref[...] + b2_ref[...]
    m = jnp.max(z, axis=1, keepdims=True)
    t = z - m
    lse = jnp.log(jnp.sum(jnp.exp(t), axis=1, keepdims=True))
    out_ref[...] = t - lse


def kernel(x, edge_index, W1, b1, W2, b2):
    n, _ = x.shape
    e = edge_index.shape[1]
    epw = e // NW
    ch = epw // C
    npad = ((n + 127) // 128) * 128

    src3 = edge_index[0].reshape(NW, ch, C)
    dst3 = edge_index[1].reshape(NW, ch, C)
    zeros_t = jnp.zeros((npad, 16), jnp.float32)
    ones_t = jnp.ones((C, 16), jnp.float32)

    pdeg = _deg_kernel(npad, ch)(dst3, zeros_t, ones_t)

    hs1, dis = pl.pallas_call(
        _tc_first,
        out_shape=(jax.ShapeDtypeStruct((n, 16), jnp.float32),
                   jax.ShapeDtypeStruct((n, 1), jnp.float32)),
    )(x, W1, pdeg)

    agg = _agg_kernel(npad, ch)
    p1 = agg(hs1, src3, dst3, zeros_t)

    hs2 = pl.pallas_call(
        _tc_mid,
        out_shape=jax.ShapeDtypeStruct((n, 16), jnp.float32),
    )(p1, hs1, dis, b1.reshape(1, 16), W2)

    p2 = agg(hs2, src3, dst3, zeros_t)

    out = pl.pallas_call(
        _tc_last,
        out_shape=jax.ShapeDtypeStruct((n, 16), jnp.float32),
    )(p2, hs2, dis, b2.reshape(1, 16))

    return out
